# speedup vs baseline: 1.5152x; 1.5152x over previous
import functools

import jax
import jax.numpy as jnp
from jax import lax
from jax.experimental import pallas as pl
from jax.experimental.pallas import tpu as pltpu

E = 8
TOPK = 2
D = 1024
I = 2048
N = 4096
A = N * TOPK
TILE = 256
PAD_A = A + E * TILE
NT = PAD_A // TILE

_INTERPRET = False


def _router_body(x_ref, gw_ref, logits_ref, rw_ref):
    x = x_ref[...]
    logits = lax.dot_general(x, gw_ref[...], (((1,), (1,)), ((), ())),
                             preferred_element_type=jnp.float32)
    logits_ref[...] = logits
    p = jax.nn.softmax(logits, axis=-1)
    blk = p.shape[0]
    iota_e = lax.broadcasted_iota(jnp.int32, (blk, E), 1)
    rank = jnp.zeros((blk, E), jnp.int32)
    for j in range(E):
        pj = p[:, j:j + 1]
        beats = (pj > p) | ((pj == p) & (j < iota_e))
        rank = rank + beats.astype(jnp.int32)
    rw_ref[...] = jnp.where(rank < TOPK, p, 0.0)


def _router_call(x, gate_W):
    blk = 1024
    return pl.pallas_call(
        _router_body,
        grid=(N // blk,),
        in_specs=[
            pl.BlockSpec((blk, D), lambda t: (t, 0)),
            pl.BlockSpec((E, D), lambda t: (0, 0)),
        ],
        out_specs=[
            pl.BlockSpec((blk, E), lambda t: (t, 0)),
            pl.BlockSpec((blk, E), lambda t: (t, 0)),
        ],
        out_shape=[
            jax.ShapeDtypeStruct((N, E), jnp.float32),
            jax.ShapeDtypeStruct((N, E), jnp.float32),
        ],
        interpret=_INTERPRET,
    )(x, gate_W)


def _dispatch(rw):
    mask = rw > 0.0
    cnt = jnp.cumsum(mask.astype(jnp.int32), axis=1)
    iota_e = jnp.arange(E, dtype=jnp.int32)[None, :]
    e1 = jnp.sum(jnp.where(mask & (cnt == 1), iota_e, 0), axis=1)
    e2 = jnp.sum(jnp.where(mask & (cnt == 2), iota_e, 0), axis=1)
    ef = jnp.stack([e1, e2], axis=1).reshape(-1)
    wf = jnp.take_along_axis(rw, jnp.stack([e1, e2], axis=1), axis=1).reshape(-1)
    onehot = (ef[:, None] == jnp.arange(E, dtype=jnp.int32)[None, :])
    ranks_all = jnp.cumsum(onehot.astype(jnp.int32), axis=0) - 1
    r = jnp.take_along_axis(ranks_all, ef[:, None], axis=1)[:, 0]
    counts = jnp.sum(onehot.astype(jnp.int32), axis=0)
    padded = ((counts + TILE - 1) // TILE) * TILE
    pstart = jnp.concatenate([jnp.zeros((1,), jnp.int32),
                              jnp.cumsum(padded).astype(jnp.int32)])
    pos = pstart[ef] + r
    token_pad = jnp.zeros((PAD_A,), jnp.int32).at[pos].set(
        jnp.arange(A, dtype=jnp.int32) // TOPK)
    wts_pad = jnp.zeros((PAD_A,), jnp.float32).at[pos].set(wf)
    tiles = jnp.arange(NT, dtype=jnp.int32)
    t_start = pstart[:-1] // TILE
    t_end = pstart[1:] // TILE
    te_gate = jnp.full((NT,), -1, jnp.int32)
    for e in range(E):
        te_gate = jnp.where((tiles >= t_start[e]) & (tiles < t_end[e]), e, te_gate)
    te_fetch = lax.associative_scan(jnp.maximum, jnp.maximum(te_gate, 0))
    return token_pad, wts_pad, te_gate, te_fetch, pos


def _mlp_body(te_gate_ref, te_fetch_ref, xs_ref, w1_ref, w3_ref, w2_ref,
              wts_ref, out_ref):
    t = pl.program_id(0)

    @pl.when(te_gate_ref[t] >= 0)
    def _():
        xs = xs_ref[...]
        g = lax.dot_general(xs, w1_ref[...], (((1,), (1,)), ((), ())),
                            preferred_element_type=jnp.float32)
        u = lax.dot_general(xs, w3_ref[...], (((1,), (1,)), ((), ())),
                            preferred_element_type=jnp.float32)
        h = g * lax.logistic(g) * u
        o = lax.dot_general(h, w2_ref[...], (((1,), (1,)), ((), ())),
                            preferred_element_type=jnp.float32)
        out_ref[...] = o * wts_ref[...]


def _mlp_call(te_gate, te_fetch, xs, w1, w3, w2, wts_pad):
    grid_spec = pltpu.PrefetchScalarGridSpec(
        num_scalar_prefetch=2,
        grid=(NT,),
        in_specs=[
            pl.BlockSpec((TILE, D), lambda t, tg, tf: (t, 0)),
            pl.BlockSpec((I, D), lambda t, tg, tf: (tf[t], 0)),
            pl.BlockSpec((I, D), lambda t, tg, tf: (tf[t], 0)),
            pl.BlockSpec((D, I), lambda t, tg, tf: (0, tf[t])),
            pl.BlockSpec((TILE, 1), lambda t, tg, tf: (t, 0)),
        ],
        out_specs=pl.BlockSpec((TILE, D), lambda t, tg, tf: (t, 0)),
    )
    return pl.pallas_call(
        _mlp_body,
        grid_spec=grid_spec,
        out_shape=jax.ShapeDtypeStruct((PAD_A, D), jnp.float32),
        interpret=_INTERPRET,
    )(te_gate, te_fetch, xs, w1, w3, w2, wts_pad[:, None])


def kernel(hidden_states, gate_W, w1, w3, w2):
    b, s, d = hidden_states.shape
    x = hidden_states.reshape(-1, d)
    logits, rw = _router_call(x, gate_W)
    token_pad, wts_pad, te_gate, te_fetch, pos = _dispatch(rw)
    xs = x[token_pad]
    outrows = _mlp_call(te_gate, te_fetch, xs, w1, w3, w2, wts_pad)
    rows = outrows[pos]
    final = rows.reshape(N, TOPK, D).sum(axis=1)
    return final.reshape(b, s, d), logits


# speedup vs baseline: 1.8854x; 1.2444x over previous
import functools

import jax
import jax.numpy as jnp
from jax import lax
from jax.experimental import pallas as pl
from jax.experimental.pallas import tpu as pltpu

E = 8
TOPK = 2
D = 1024
I = 2048
N = 4096
A = N * TOPK
TILE = 256
PAD_A = A + E * TILE
NT = PAD_A // TILE
RBLK = 1024

_INTERPRET = False


def _router_body(x_ref, gw_ref, lt_ref, logits_ref, ea_ref, eb_ref, wa_ref,
                 wb_ref, ra_ref, rb_ref, counts_ref, carry_ref):
    t = pl.program_id(0)
    x = x_ref[...]
    logits = lax.dot_general(x, gw_ref[...], (((1,), (1,)), ((), ())),
                             preferred_element_type=jnp.float32)
    logits_ref[...] = logits
    p = jax.nn.softmax(logits, axis=-1)
    iota_e = lax.broadcasted_iota(jnp.int32, (RBLK, E), 1)
    rank = jnp.zeros((RBLK, E), jnp.int32)
    for j in range(E):
        pj = p[:, j:j + 1]
        beats = (pj > p) | ((pj == p) & (j < iota_e))
        rank = rank + beats.astype(jnp.int32)
    sel = rank < TOPK
    acc = jnp.zeros((RBLK, 1), jnp.int32)
    cols = []
    for j in range(E):
        acc = acc + sel[:, j:j + 1].astype(jnp.int32)
        cols.append(acc)
    cnt = jnp.concatenate(cols, axis=1)
    first = sel & (cnt == 1)
    second = sel & (cnt == 2)
    ea = jnp.sum(jnp.where(first, iota_e, 0), axis=1, keepdims=True)
    eb = jnp.sum(jnp.where(second, iota_e, 0), axis=1, keepdims=True)
    ea_ref[...] = ea[:, 0]
    eb_ref[...] = eb[:, 0]
    wa_ref[...] = jnp.sum(jnp.where(first, p, 0.0), axis=1)
    wb_ref[...] = jnp.sum(jnp.where(second, p, 0.0), axis=1)
    self = sel.astype(jnp.float32)
    cumexcl = lax.dot_general(lt_ref[...], self, (((1,), (0,)), ((), ())),
                              preferred_element_type=jnp.float32)

    @pl.when(t == 0)
    def _():
        carry_ref[...] = jnp.zeros((1, E), jnp.float32)

    carry = carry_ref[...]
    cumexcl = cumexcl + carry
    ra_ref[...] = jnp.sum(
        jnp.where(iota_e == ea, cumexcl, 0.0), axis=1).astype(jnp.int32)
    rb_ref[...] = jnp.sum(
        jnp.where(iota_e == eb, cumexcl, 0.0), axis=1).astype(jnp.int32)
    new_carry = carry + jnp.sum(self, axis=0, keepdims=True)
    carry_ref[...] = new_carry
    counts_ref[...] = new_carry[0]


def _router_call(x, gate_W, ltri):
    return pl.pallas_call(
        _router_body,
        grid=(N // RBLK,),
        in_specs=[
            pl.BlockSpec((RBLK, D), lambda t: (t, 0)),
            pl.BlockSpec((E, D), lambda t: (0, 0)),
            pl.BlockSpec((RBLK, RBLK), lambda t: (0, 0)),
        ],
        out_specs=[
            pl.BlockSpec((RBLK, E), lambda t: (t, 0)),
            pl.BlockSpec((RBLK,), lambda t: (t,)),
            pl.BlockSpec((RBLK,), lambda t: (t,)),
            pl.BlockSpec((RBLK,), lambda t: (t,)),
            pl.BlockSpec((RBLK,), lambda t: (t,)),
            pl.BlockSpec((RBLK,), lambda t: (t,)),
            pl.BlockSpec((RBLK,), lambda t: (t,)),
            pl.BlockSpec((E,), lambda t: (0,)),
        ],
        out_shape=[
            jax.ShapeDtypeStruct((N, E), jnp.float32),
            jax.ShapeDtypeStruct((N,), jnp.int32),
            jax.ShapeDtypeStruct((N,), jnp.int32),
            jax.ShapeDtypeStruct((N,), jnp.float32),
            jax.ShapeDtypeStruct((N,), jnp.float32),
            jax.ShapeDtypeStruct((N,), jnp.int32),
            jax.ShapeDtypeStruct((N,), jnp.int32),
            jax.ShapeDtypeStruct((E,), jnp.float32),
        ],
        scratch_shapes=[pltpu.VMEM((1, E), jnp.float32)],
        interpret=_INTERPRET,
    )(x, gate_W, ltri)


def _mlp_body(te_gate_ref, te_fetch_ref, xs_ref, w1_ref, w3_ref, w2_ref,
              out_ref):
    t = pl.program_id(0)

    @pl.when(te_gate_ref[t] >= 0)
    def _():
        xs = xs_ref[...]
        g = lax.dot_general(xs, w1_ref[...], (((1,), (1,)), ((), ())),
                            preferred_element_type=jnp.float32)
        u = lax.dot_general(xs, w3_ref[...], (((1,), (1,)), ((), ())),
                            preferred_element_type=jnp.float32)
        h = g * lax.logistic(g) * u
        out_ref[...] = lax.dot_general(h, w2_ref[...], (((1,), (1,)), ((), ())),
                                       preferred_element_type=jnp.float32)


def _mlp_call(te_gate, te_fetch, xs, w1, w3, w2):
    grid_spec = pltpu.PrefetchScalarGridSpec(
        num_scalar_prefetch=2,
        grid=(NT,),
        in_specs=[
            pl.BlockSpec((TILE, D), lambda t, tg, tf: (t, 0)),
            pl.BlockSpec((I, D), lambda t, tg, tf: (tf[t], 0)),
            pl.BlockSpec((I, D), lambda t, tg, tf: (tf[t], 0)),
            pl.BlockSpec((D, I), lambda t, tg, tf: (0, tf[t])),
        ],
        out_specs=pl.BlockSpec((TILE, D), lambda t, tg, tf: (t, 0)),
    )
    return pl.pallas_call(
        _mlp_body,
        grid_spec=grid_spec,
        out_shape=jax.ShapeDtypeStruct((PAD_A, D), jnp.float32),
        interpret=_INTERPRET,
    )(te_gate, te_fetch, xs, w1, w3, w2)


def kernel(hidden_states, gate_W, w1, w3, w2):
    b, s, d = hidden_states.shape
    x = hidden_states.reshape(-1, d)
    ltri = jnp.tril(jnp.ones((RBLK, RBLK), jnp.float32), -1)
    logits, ea, eb, wa, wb, ra, rb, counts = _router_call(x, gate_W, ltri)
    counts = counts.astype(jnp.int32)
    padded = ((counts + TILE - 1) // TILE) * TILE
    pstart = jnp.concatenate([jnp.zeros((1,), jnp.int32),
                              jnp.cumsum(padded).astype(jnp.int32)])
    pos_a = pstart[ea] + ra
    pos_b = pstart[eb] + rb
    tiles = jnp.arange(NT, dtype=jnp.int32)
    t_start = pstart[:-1] // TILE
    t_end = pstart[1:] // TILE
    te_gate = jnp.full((NT,), -1, jnp.int32)
    for e in range(E):
        te_gate = jnp.where((tiles >= t_start[e]) & (tiles < t_end[e]), e, te_gate)
    te_fetch = lax.associative_scan(jnp.maximum, jnp.maximum(te_gate, 0))

    xs = jnp.zeros((PAD_A, D), jnp.float32).at[pos_a].set(x).at[pos_b].set(x)
    outrows = _mlp_call(te_gate, te_fetch, xs, w1, w3, w2)
    final = outrows[pos_a] * wa[:, None] + outrows[pos_b] * wb[:, None]
    return final.reshape(b, s, d), logits


# speedup vs baseline: 2.1361x; 1.1329x over previous
import functools

import jax
import jax.numpy as jnp
from jax import lax
from jax.experimental import pallas as pl
from jax.experimental.pallas import tpu as pltpu
from jax.experimental.pallas import tpu_sc as plsc

E = 8
TOPK = 2
D = 1024
I = 2048
N = 4096
A = N * TOPK
TILE = 256
PAD_A = A + E * TILE
NT = PAD_A // TILE
RBLK = 1024

_INTERPRET = False


def _router_body(x_ref, gw_ref, lt_ref, logits_ref, ea_ref, eb_ref, wa_ref,
                 wb_ref, ra_ref, rb_ref, counts_ref, carry_ref):
    t = pl.program_id(0)
    x = x_ref[...]
    logits = lax.dot_general(x, gw_ref[...], (((1,), (1,)), ((), ())),
                             preferred_element_type=jnp.float32)
    logits_ref[...] = logits
    p = jax.nn.softmax(logits, axis=-1)
    iota_e = lax.broadcasted_iota(jnp.int32, (RBLK, E), 1)
    rank = jnp.zeros((RBLK, E), jnp.int32)
    for j in range(E):
        pj = p[:, j:j + 1]
        beats = (pj > p) | ((pj == p) & (j < iota_e))
        rank = rank + beats.astype(jnp.int32)
    sel = rank < TOPK
    acc = jnp.zeros((RBLK, 1), jnp.int32)
    cols = []
    for j in range(E):
        acc = acc + sel[:, j:j + 1].astype(jnp.int32)
        cols.append(acc)
    cnt = jnp.concatenate(cols, axis=1)
    first = sel & (cnt == 1)
    second = sel & (cnt == 2)
    ea = jnp.sum(jnp.where(first, iota_e, 0), axis=1, keepdims=True)
    eb = jnp.sum(jnp.where(second, iota_e, 0), axis=1, keepdims=True)
    ea_ref[...] = ea[:, 0]
    eb_ref[...] = eb[:, 0]
    wa_ref[...] = jnp.sum(jnp.where(first, p, 0.0), axis=1)
    wb_ref[...] = jnp.sum(jnp.where(second, p, 0.0), axis=1)
    self = sel.astype(jnp.float32)
    cumexcl = lax.dot_general(lt_ref[...], self, (((1,), (0,)), ((), ())),
                              preferred_element_type=jnp.float32)

    @pl.when(t == 0)
    def _():
        carry_ref[...] = jnp.zeros((1, E), jnp.float32)

    carry = carry_ref[...]
    cumexcl = cumexcl + carry
    ra_ref[...] = jnp.sum(
        jnp.where(iota_e == ea, cumexcl, 0.0), axis=1).astype(jnp.int32)
    rb_ref[...] = jnp.sum(
        jnp.where(iota_e == eb, cumexcl, 0.0), axis=1).astype(jnp.int32)
    new_carry = carry + jnp.sum(self, axis=0, keepdims=True)
    carry_ref[...] = new_carry
    counts_ref[...] = new_carry[0]


def _router_call(x, gate_W, ltri):
    return pl.pallas_call(
        _router_body,
        grid=(N // RBLK,),
        in_specs=[
            pl.BlockSpec((RBLK, D), lambda t: (t, 0)),
            pl.BlockSpec((E, D), lambda t: (0, 0)),
            pl.BlockSpec((RBLK, RBLK), lambda t: (0, 0)),
        ],
        out_specs=[
            pl.BlockSpec((RBLK, E), lambda t: (t, 0)),
            pl.BlockSpec((RBLK,), lambda t: (t,)),
            pl.BlockSpec((RBLK,), lambda t: (t,)),
            pl.BlockSpec((RBLK,), lambda t: (t,)),
            pl.BlockSpec((RBLK,), lambda t: (t,)),
            pl.BlockSpec((RBLK,), lambda t: (t,)),
            pl.BlockSpec((RBLK,), lambda t: (t,)),
            pl.BlockSpec((E,), lambda t: (0,)),
        ],
        out_shape=[
            jax.ShapeDtypeStruct((N, E), jnp.float32),
            jax.ShapeDtypeStruct((N,), jnp.int32),
            jax.ShapeDtypeStruct((N,), jnp.int32),
            jax.ShapeDtypeStruct((N,), jnp.float32),
            jax.ShapeDtypeStruct((N,), jnp.float32),
            jax.ShapeDtypeStruct((N,), jnp.int32),
            jax.ShapeDtypeStruct((N,), jnp.int32),
            jax.ShapeDtypeStruct((E,), jnp.float32),
        ],
        scratch_shapes=[pltpu.VMEM((1, E), jnp.float32)],
        interpret=_INTERPRET,
    )(x, gate_W, ltri)


def _mlp_body(te_gate_ref, te_fetch_ref, xs_ref, w1_ref, w3_ref, w2_ref,
              out_ref):
    t = pl.program_id(0)

    @pl.when(te_gate_ref[t] >= 0)
    def _():
        xs = xs_ref[...]
        g = lax.dot_general(xs, w1_ref[...], (((1,), (1,)), ((), ())),
                            preferred_element_type=jnp.float32)
        u = lax.dot_general(xs, w3_ref[...], (((1,), (1,)), ((), ())),
                            preferred_element_type=jnp.float32)
        h = g * lax.logistic(g) * u
        out_ref[...] = lax.dot_general(h, w2_ref[...], (((1,), (1,)), ((), ())),
                                       preferred_element_type=jnp.float32)


def _mlp_call(te_gate, te_fetch, xs, w1, w3, w2):
    grid_spec = pltpu.PrefetchScalarGridSpec(
        num_scalar_prefetch=2,
        grid=(NT,),
        in_specs=[
            pl.BlockSpec((TILE, D), lambda t, tg, tf: (t, 0)),
            pl.BlockSpec((I, D), lambda t, tg, tf: (tf[t], 0)),
            pl.BlockSpec((I, D), lambda t, tg, tf: (tf[t], 0)),
            pl.BlockSpec((D, I), lambda t, tg, tf: (0, tf[t])),
        ],
        out_specs=pl.BlockSpec((TILE, D), lambda t, tg, tf: (t, 0)),
    )
    return pl.pallas_call(
        _mlp_body,
        grid_spec=grid_spec,
        out_shape=jax.ShapeDtypeStruct((PAD_A, D), jnp.float32),
        interpret=_INTERPRET,
    )(te_gate, te_fetch, xs, w1, w3, w2)


NW = 32
TPW = N // NW
DCHUNK = 64
CCHUNK = 32

_SC_MESH = dict(core_axis_name="c", subcore_axis_name="s")


def _sc_wid():
    return lax.axis_index("s") * 2 + lax.axis_index("c")


@functools.partial(
    pl.kernel,
    mesh=plsc.VectorSubcoreMesh(**_SC_MESH),
    out_type=jax.ShapeDtypeStruct((PAD_A, D), jnp.float32),
    scratch_types=[
        pltpu.VMEM((DCHUNK,), jnp.int32),
        pltpu.VMEM((DCHUNK,), jnp.int32),
        pltpu.VMEM((DCHUNK, D), jnp.float32),
        pltpu.SemaphoreType.DMA,
    ],
)
def _sc_dispatch(x_hbm, pa_hbm, pb_hbm, xs_hbm, ia_v, ib_v, rows_v, sem):
    base = _sc_wid() * TPW

    def chunk(c, carry):
        t0 = base + c * DCHUNK
        pltpu.sync_copy(x_hbm.at[pl.ds(t0, DCHUNK)], rows_v)
        pltpu.sync_copy(pa_hbm.at[pl.ds(t0, DCHUNK)], ia_v)
        pltpu.sync_copy(pb_hbm.at[pl.ds(t0, DCHUNK)], ib_v)
        pltpu.async_copy(rows_v, xs_hbm.at[ia_v], sem).wait()
        pltpu.async_copy(rows_v, xs_hbm.at[ib_v], sem).wait()
        return carry

    lax.fori_loop(0, TPW // DCHUNK, chunk, 0)


@functools.partial(
    pl.kernel,
    mesh=plsc.VectorSubcoreMesh(**_SC_MESH),
    out_type=jax.ShapeDtypeStruct((N, D), jnp.float32),
    scratch_types=[
        pltpu.VMEM((2 * CCHUNK,), jnp.int32),
        pltpu.VMEM((2 * CCHUNK + 16,), jnp.float32),
        pltpu.VMEM((2 * CCHUNK, D), jnp.float32),
        pltpu.VMEM((CCHUNK, D), jnp.float32),
        pltpu.SemaphoreType.DMA,
    ],
)
def _sc_combine(rows_hbm, pos_hbm, wf_hbm, out_hbm, idx_v, wf_v, rows_v,
                acc_v, sem):
    base = _sc_wid() * TPW

    def chunk(c, carry):
        t0 = base + c * CCHUNK
        pltpu.sync_copy(pos_hbm.at[pl.ds(2 * t0, 2 * CCHUNK)], idx_v)
        pltpu.sync_copy(wf_hbm.at[pl.ds(2 * t0, 2 * CCHUNK)],
                        wf_v.at[pl.ds(0, 2 * CCHUNK)])
        pltpu.async_copy(rows_hbm.at[idx_v], rows_v, sem).wait()

        def tok(i, c2):
            wpair = wf_v[pl.ds(2 * i, 16)]
            w0 = wpair[0]
            w1 = wpair[1]

            def vec(j, c3):
                sl = pl.ds(j * 16, 16)
                acc_v[i, sl] = rows_v[2 * i, sl] * w0 + rows_v[2 * i + 1, sl] * w1
                return c3

            return lax.fori_loop(0, D // 16, vec, c2)

        lax.fori_loop(0, CCHUNK, tok, 0)
        pltpu.sync_copy(acc_v, out_hbm.at[pl.ds(t0, CCHUNK)])
        return carry

    lax.fori_loop(0, TPW // CCHUNK, chunk, 0)


def kernel(hidden_states, gate_W, w1, w3, w2):
    b, s, d = hidden_states.shape
    x = hidden_states.reshape(-1, d)
    ltri = jnp.tril(jnp.ones((RBLK, RBLK), jnp.float32), -1)
    logits, ea, eb, wa, wb, ra, rb, counts = _router_call(x, gate_W, ltri)
    counts = counts.astype(jnp.int32)
    padded = ((counts + TILE - 1) // TILE) * TILE
    pstart = jnp.concatenate([jnp.zeros((1,), jnp.int32),
                              jnp.cumsum(padded).astype(jnp.int32)])
    pos_a = pstart[ea] + ra
    pos_b = pstart[eb] + rb
    tiles = jnp.arange(NT, dtype=jnp.int32)
    t_start = pstart[:-1] // TILE
    t_end = pstart[1:] // TILE
    te_gate = jnp.full((NT,), -1, jnp.int32)
    for e in range(E):
        te_gate = jnp.where((tiles >= t_start[e]) & (tiles < t_end[e]), e, te_gate)
    te_fetch = lax.associative_scan(jnp.maximum, jnp.maximum(te_gate, 0))

    xs = _sc_dispatch(x, pos_a, pos_b)
    outrows = _mlp_call(te_gate, te_fetch, xs, w1, w3, w2)
    pos = jnp.stack([pos_a, pos_b], axis=1).reshape(-1)
    wf = jnp.stack([wa, wb], axis=1).reshape(-1)
    final = _sc_combine(outrows, pos, wf)
    return final.reshape(b, s, d), logits


# speedup vs baseline: 2.1609x; 1.0116x over previous
import functools

import jax
import jax.numpy as jnp
from jax import lax
from jax.experimental import pallas as pl
from jax.experimental.pallas import tpu as pltpu
from jax.experimental.pallas import tpu_sc as plsc

E = 8
TOPK = 2
D = 1024
I = 2048
N = 4096
A = N * TOPK
TILE = 256
PAD_A = A + E * TILE
NT = PAD_A // TILE
RBLK = 1024

_INTERPRET = False


def _router_body(x_ref, gw_ref, lt_ref, logits_ref, ea_ref, eb_ref, wa_ref,
                 wb_ref, ra_ref, rb_ref, counts_ref, carry_ref):
    t = pl.program_id(0)
    x = x_ref[...]
    logits = lax.dot_general(x, gw_ref[...], (((1,), (1,)), ((), ())),
                             preferred_element_type=jnp.float32)
    logits_ref[...] = logits
    p = jax.nn.softmax(logits, axis=-1)
    iota_e = lax.broadcasted_iota(jnp.int32, (RBLK, E), 1)
    rank = jnp.zeros((RBLK, E), jnp.int32)
    for j in range(E):
        pj = p[:, j:j + 1]
        beats = (pj > p) | ((pj == p) & (j < iota_e))
        rank = rank + beats.astype(jnp.int32)
    sel = rank < TOPK
    acc = jnp.zeros((RBLK, 1), jnp.int32)
    cols = []
    for j in range(E):
        acc = acc + sel[:, j:j + 1].astype(jnp.int32)
        cols.append(acc)
    cnt = jnp.concatenate(cols, axis=1)
    first = sel & (cnt == 1)
    second = sel & (cnt == 2)
    ea = jnp.sum(jnp.where(first, iota_e, 0), axis=1, keepdims=True)
    eb = jnp.sum(jnp.where(second, iota_e, 0), axis=1, keepdims=True)
    ea_ref[...] = ea[:, 0]
    eb_ref[...] = eb[:, 0]
    wa_ref[...] = jnp.sum(jnp.where(first, p, 0.0), axis=1)
    wb_ref[...] = jnp.sum(jnp.where(second, p, 0.0), axis=1)
    self = sel.astype(jnp.float32)
    cumexcl = lax.dot_general(lt_ref[...], self, (((1,), (0,)), ((), ())),
                              preferred_element_type=jnp.float32)

    @pl.when(t == 0)
    def _():
        carry_ref[...] = jnp.zeros((1, E), jnp.float32)

    carry = carry_ref[...]
    cumexcl = cumexcl + carry
    ra_ref[...] = jnp.sum(
        jnp.where(iota_e == ea, cumexcl, 0.0), axis=1).astype(jnp.int32)
    rb_ref[...] = jnp.sum(
        jnp.where(iota_e == eb, cumexcl, 0.0), axis=1).astype(jnp.int32)
    new_carry = carry + jnp.sum(self, axis=0, keepdims=True)
    carry_ref[...] = new_carry
    counts_ref[...] = new_carry[0]


def _router_call(x, gate_W, ltri):
    return pl.pallas_call(
        _router_body,
        grid=(N // RBLK,),
        in_specs=[
            pl.BlockSpec((RBLK, D), lambda t: (t, 0)),
            pl.BlockSpec((E, D), lambda t: (0, 0)),
            pl.BlockSpec((RBLK, RBLK), lambda t: (0, 0)),
        ],
        out_specs=[
            pl.BlockSpec((RBLK, E), lambda t: (t, 0)),
            pl.BlockSpec((RBLK,), lambda t: (t,)),
            pl.BlockSpec((RBLK,), lambda t: (t,)),
            pl.BlockSpec((RBLK,), lambda t: (t,)),
            pl.BlockSpec((RBLK,), lambda t: (t,)),
            pl.BlockSpec((RBLK,), lambda t: (t,)),
            pl.BlockSpec((RBLK,), lambda t: (t,)),
            pl.BlockSpec((E,), lambda t: (0,)),
        ],
        out_shape=[
            jax.ShapeDtypeStruct((N, E), jnp.float32),
            jax.ShapeDtypeStruct((N,), jnp.int32),
            jax.ShapeDtypeStruct((N,), jnp.int32),
            jax.ShapeDtypeStruct((N,), jnp.float32),
            jax.ShapeDtypeStruct((N,), jnp.float32),
            jax.ShapeDtypeStruct((N,), jnp.int32),
            jax.ShapeDtypeStruct((N,), jnp.int32),
            jax.ShapeDtypeStruct((E,), jnp.float32),
        ],
        scratch_shapes=[pltpu.VMEM((1, E), jnp.float32)],
        interpret=_INTERPRET,
    )(x, gate_W, ltri)


def _mlp_body(te_gate_ref, te_fetch_ref, xs_ref, w1_ref, w3_ref, w2_ref,
              wt_ref, out_ref):
    t = pl.program_id(0)

    @pl.when(te_gate_ref[t] >= 0)
    def _():
        xs = xs_ref[...]
        g = lax.dot_general(xs, w1_ref[...], (((1,), (1,)), ((), ())),
                            preferred_element_type=jnp.float32)
        u = lax.dot_general(xs, w3_ref[...], (((1,), (1,)), ((), ())),
                            preferred_element_type=jnp.float32)
        h = g * lax.logistic(g) * u
        o = lax.dot_general(h, w2_ref[...], (((1,), (1,)), ((), ())),
                            preferred_element_type=jnp.float32)
        out_ref[...] = o * wt_ref[:, :1]


def _mlp_call(te_gate, te_fetch, xs, w1, w3, w2, wts16):
    grid_spec = pltpu.PrefetchScalarGridSpec(
        num_scalar_prefetch=2,
        grid=(NT,),
        in_specs=[
            pl.BlockSpec((TILE, D), lambda t, tg, tf: (t, 0)),
            pl.BlockSpec((I, D), lambda t, tg, tf: (tf[t], 0)),
            pl.BlockSpec((I, D), lambda t, tg, tf: (tf[t], 0)),
            pl.BlockSpec((D, I), lambda t, tg, tf: (0, tf[t])),
            pl.BlockSpec((TILE, 128), lambda t, tg, tf: (t, 0)),
        ],
        out_specs=pl.BlockSpec((TILE, D), lambda t, tg, tf: (t, 0)),
    )
    return pl.pallas_call(
        _mlp_body,
        grid_spec=grid_spec,
        out_shape=jax.ShapeDtypeStruct((PAD_A, D), jnp.float32),
        interpret=_INTERPRET,
    )(te_gate, te_fetch, xs, w1, w3, w2, wts16)


NW = 32
TPW = N // NW
DCHUNK = 64
CCHUNK = 32

_SC_MESH = dict(core_axis_name="c", subcore_axis_name="s")


def _sc_wid():
    return lax.axis_index("s") * 2 + lax.axis_index("c")


@functools.partial(
    pl.kernel,
    mesh=plsc.VectorSubcoreMesh(**_SC_MESH),
    out_type=[
        jax.ShapeDtypeStruct((PAD_A, D), jnp.float32),
        jax.ShapeDtypeStruct((PAD_A, 128), jnp.float32),
    ],
    scratch_types=[
        pltpu.VMEM((DCHUNK,), jnp.int32),
        pltpu.VMEM((DCHUNK,), jnp.int32),
        pltpu.VMEM((DCHUNK, D), jnp.float32),
        pltpu.VMEM((DCHUNK, 128), jnp.float32),
        pltpu.VMEM((DCHUNK, 128), jnp.float32),
        pltpu.SemaphoreType.DMA,
    ],
)
def _sc_dispatch(x_hbm, pa_hbm, pb_hbm, wa_hbm, wb_hbm, xs_hbm, wts_hbm,
                 ia_v, ib_v, rows_v, wa_v, wb_v, sem):
    base = _sc_wid() * TPW

    def chunk(c, carry):
        t0 = base + c * DCHUNK
        pltpu.sync_copy(x_hbm.at[pl.ds(t0, DCHUNK)], rows_v)
        pltpu.sync_copy(pa_hbm.at[pl.ds(t0, DCHUNK)], ia_v)
        pltpu.sync_copy(pb_hbm.at[pl.ds(t0, DCHUNK)], ib_v)
        pltpu.sync_copy(wa_hbm.at[pl.ds(t0, DCHUNK)], wa_v)
        pltpu.sync_copy(wb_hbm.at[pl.ds(t0, DCHUNK)], wb_v)
        pltpu.async_copy(rows_v, xs_hbm.at[ia_v], sem).wait()
        pltpu.async_copy(rows_v, xs_hbm.at[ib_v], sem).wait()
        pltpu.async_copy(wa_v, wts_hbm.at[ia_v], sem).wait()
        pltpu.async_copy(wb_v, wts_hbm.at[ib_v], sem).wait()
        return carry

    lax.fori_loop(0, TPW // DCHUNK, chunk, 0)


@functools.partial(
    pl.kernel,
    mesh=plsc.VectorSubcoreMesh(**_SC_MESH),
    out_type=jax.ShapeDtypeStruct((N, D), jnp.float32),
    scratch_types=[
        pltpu.VMEM((2 * CCHUNK,), jnp.int32),
        pltpu.VMEM((2 * CCHUNK, D), jnp.float32),
        pltpu.VMEM((CCHUNK, D), jnp.float32),
        pltpu.SemaphoreType.DMA,
    ],
)
def _sc_combine(rows_hbm, pos_hbm, out_hbm, idx_v, rows_v, acc_v, sem):
    base = _sc_wid() * TPW

    def chunk(c, carry):
        t0 = base + c * CCHUNK
        pltpu.sync_copy(pos_hbm.at[pl.ds(2 * t0, 2 * CCHUNK)], idx_v)
        pltpu.async_copy(rows_hbm.at[idx_v], rows_v, sem).wait()

        def tok(i, c2):
            for j in range(D // 16):
                sl = pl.ds(j * 16, 16)
                acc_v[i, sl] = rows_v[2 * i, sl] + rows_v[2 * i + 1, sl]
            return c2

        lax.fori_loop(0, CCHUNK, tok, 0)
        pltpu.sync_copy(acc_v, out_hbm.at[pl.ds(t0, CCHUNK)])
        return carry

    lax.fori_loop(0, TPW // CCHUNK, chunk, 0)


def kernel(hidden_states, gate_W, w1, w3, w2):
    b, s, d = hidden_states.shape
    x = hidden_states.reshape(-1, d)
    ltri = jnp.tril(jnp.ones((RBLK, RBLK), jnp.float32), -1)
    logits, ea, eb, wa, wb, ra, rb, counts = _router_call(x, gate_W, ltri)
    counts = counts.astype(jnp.int32)
    padded = ((counts + TILE - 1) // TILE) * TILE
    pstart = jnp.concatenate([jnp.zeros((1,), jnp.int32),
                              jnp.cumsum(padded).astype(jnp.int32)])
    pos_a = pstart[ea] + ra
    pos_b = pstart[eb] + rb
    tiles = jnp.arange(NT, dtype=jnp.int32)
    t_start = pstart[:-1] // TILE
    t_end = pstart[1:] // TILE
    te_gate = jnp.full((NT,), -1, jnp.int32)
    for e in range(E):
        te_gate = jnp.where((tiles >= t_start[e]) & (tiles < t_end[e]), e, te_gate)
    te_fetch = lax.associative_scan(jnp.maximum, jnp.maximum(te_gate, 0))

    ones16 = jnp.ones((1, 128), jnp.float32)
    xs, wts16 = _sc_dispatch(x, pos_a, pos_b, wa[:, None] * ones16,
                             wb[:, None] * ones16)
    outrows = _mlp_call(te_gate, te_fetch, xs, w1, w3, w2, wts16)
    pos = jnp.stack([pos_a, pos_b], axis=1).reshape(-1)
    final = _sc_combine(outrows, pos)
    return final.reshape(b, s, d), logits


# speedup vs baseline: 2.6375x; 1.2205x over previous
import functools

import jax
import jax.numpy as jnp
from jax import lax
from jax.experimental import pallas as pl
from jax.experimental.pallas import tpu as pltpu
from jax.experimental.pallas import tpu_sc as plsc

E = 8
TOPK = 2
D = 1024
I = 2048
N = 4096
A = N * TOPK
TILE = 256
PAD_A = A + E * TILE
NT = PAD_A // TILE
RBLK = 1024

_INTERPRET = False


def _router_body(x_ref, gw_ref, ut_ref, logits_ref, ea_ref, eb_ref, wa16_ref,
                 wb16_ref, ra_ref, rb_ref, counts_ref, carry_ref):
    t = pl.program_id(0)
    x = x_ref[...]
    logits_ref[...] = lax.dot_general(x, gw_ref[...], (((1,), (1,)), ((), ())),
                                      preferred_element_type=jnp.float32)
    lt = lax.dot_general(gw_ref[...], x, (((1,), (1,)), ((), ())),
                         preferred_element_type=jnp.float32)
    pt = jax.nn.softmax(lt, axis=0)
    iota_s = lax.broadcasted_iota(jnp.int32, (E, RBLK), 0)
    rank = jnp.zeros((E, RBLK), jnp.int32)
    for j in range(E):
        pj = pt[j:j + 1, :]
        beats = (pj > pt) | ((pj == pt) & (j < iota_s))
        rank = rank + beats.astype(jnp.int32)
    sel = rank < TOPK
    acc = jnp.zeros((1, RBLK), jnp.int32)
    rows = []
    for j in range(E):
        acc = acc + sel[j:j + 1, :].astype(jnp.int32)
        rows.append(acc)
    cnt = jnp.concatenate(rows, axis=0)
    first = sel & (cnt == 1)
    second = sel & (cnt == 2)
    ea = jnp.sum(jnp.where(first, iota_s, 0), axis=0, keepdims=True)
    eb = jnp.sum(jnp.where(second, iota_s, 0), axis=0, keepdims=True)
    ea_ref[...] = ea[0]
    eb_ref[...] = eb[0]
    wa = jnp.sum(jnp.where(first, pt, 0.0), axis=0)
    wb = jnp.sum(jnp.where(second, pt, 0.0), axis=0)
    ones_l = jnp.ones((1, 128), jnp.float32)
    wa16_ref[...] = wa[:, None] * ones_l
    wb16_ref[...] = wb[:, None] * ones_l
    self = sel.astype(jnp.float32)
    cumexcl = lax.dot_general(self, ut_ref[...], (((1,), (0,)), ((), ())),
                              preferred_element_type=jnp.float32)

    @pl.when(t == 0)
    def _():
        carry_ref[...] = jnp.zeros((E, 128), jnp.float32)

    carry = carry_ref[...][:, :1]
    cumexcl = cumexcl + carry
    ra_ref[...] = jnp.sum(
        jnp.where(iota_s == ea, cumexcl, 0.0), axis=0).astype(jnp.int32)
    rb_ref[...] = jnp.sum(
        jnp.where(iota_s == eb, cumexcl, 0.0), axis=0).astype(jnp.int32)
    new_carry = carry + jnp.sum(self, axis=1, keepdims=True)
    carry_ref[...] = new_carry * jnp.ones((1, 128), jnp.float32)
    counts_ref[...] = new_carry[:, 0]


def _router_call(x, gate_W, utri):
    return pl.pallas_call(
        _router_body,
        grid=(N // RBLK,),
        in_specs=[
            pl.BlockSpec((RBLK, D), lambda t: (t, 0)),
            pl.BlockSpec((E, D), lambda t: (0, 0)),
            pl.BlockSpec((RBLK, RBLK), lambda t: (0, 0)),
        ],
        out_specs=[
            pl.BlockSpec((RBLK, E), lambda t: (t, 0)),
            pl.BlockSpec((RBLK,), lambda t: (t,)),
            pl.BlockSpec((RBLK,), lambda t: (t,)),
            pl.BlockSpec((RBLK, 128), lambda t: (t, 0)),
            pl.BlockSpec((RBLK, 128), lambda t: (t, 0)),
            pl.BlockSpec((RBLK,), lambda t: (t,)),
            pl.BlockSpec((RBLK,), lambda t: (t,)),
            pl.BlockSpec((E,), lambda t: (0,)),
        ],
        out_shape=[
            jax.ShapeDtypeStruct((N, E), jnp.float32),
            jax.ShapeDtypeStruct((N,), jnp.int32),
            jax.ShapeDtypeStruct((N,), jnp.int32),
            jax.ShapeDtypeStruct((N, 128), jnp.float32),
            jax.ShapeDtypeStruct((N, 128), jnp.float32),
            jax.ShapeDtypeStruct((N,), jnp.int32),
            jax.ShapeDtypeStruct((N,), jnp.int32),
            jax.ShapeDtypeStruct((E,), jnp.float32),
        ],
        scratch_shapes=[pltpu.VMEM((E, 128), jnp.float32)],
        interpret=_INTERPRET,
    )(x, gate_W, utri)


def _tile_expert(t, ps):
    s = jnp.int32(0)
    for e in range(1, E + 1):
        s = s + (ps[e] <= t * TILE).astype(jnp.int32)
    return jnp.minimum(s, E - 1)


def _mlp_body(ps_ref, xs_ref, w1_ref, w3_ref, w2_ref, wt_ref, out_ref):
    t = pl.program_id(0)

    @pl.when(t * TILE < ps_ref[E])
    def _():
        xs = xs_ref[...]
        g = lax.dot_general(xs, w1_ref[...], (((1,), (1,)), ((), ())),
                            preferred_element_type=jnp.float32)
        u = lax.dot_general(xs, w3_ref[...], (((1,), (1,)), ((), ())),
                            preferred_element_type=jnp.float32)
        h = g * lax.logistic(g) * u
        o = lax.dot_general(h, w2_ref[...], (((1,), (1,)), ((), ())),
                            preferred_element_type=jnp.float32)
        out_ref[...] = o * wt_ref[:, :1]


def _mlp_call(pstart, xs, w1, w3, w2, wts16):
    grid_spec = pltpu.PrefetchScalarGridSpec(
        num_scalar_prefetch=1,
        grid=(NT,),
        in_specs=[
            pl.BlockSpec((TILE, D), lambda t, ps: (t, 0)),
            pl.BlockSpec((I, D), lambda t, ps: (_tile_expert(t, ps), 0)),
            pl.BlockSpec((I, D), lambda t, ps: (_tile_expert(t, ps), 0)),
            pl.BlockSpec((D, I), lambda t, ps: (0, _tile_expert(t, ps))),
            pl.BlockSpec((TILE, 128), lambda t, ps: (t, 0)),
        ],
        out_specs=pl.BlockSpec((TILE, D), lambda t, ps: (t, 0)),
    )
    return pl.pallas_call(
        _mlp_body,
        grid_spec=grid_spec,
        out_shape=jax.ShapeDtypeStruct((PAD_A, D), jnp.float32),
        interpret=_INTERPRET,
    )(pstart, xs, w1, w3, w2, wts16)


NW = 32
TPW = N // NW
DCHUNK = 64
CCHUNK = 32

_SC_MESH = dict(core_axis_name="c", subcore_axis_name="s")


def _sc_wid():
    return lax.axis_index("s") * 2 + lax.axis_index("c")


@functools.cache
def _sc_dispatch_kernel():
    return functools.partial(
        pl.kernel,
        mesh=plsc.VectorSubcoreMesh(**_SC_MESH),
        out_type=[
            jax.ShapeDtypeStruct((PAD_A, D), jnp.float32),
            jax.ShapeDtypeStruct((PAD_A, 128), jnp.float32),
        ],
        scratch_types=[
            pltpu.VMEM((DCHUNK,), jnp.int32),
            pltpu.VMEM((DCHUNK,), jnp.int32),
            pltpu.VMEM((DCHUNK, D), jnp.float32),
            pltpu.VMEM((DCHUNK, 128), jnp.float32),
            pltpu.VMEM((DCHUNK, 128), jnp.float32),
            pltpu.SemaphoreType.DMA,
        ],
    )(_sc_dispatch_body)


def _sc_dispatch(x, pos_a, pos_b, wa16, wb16):
    return _sc_dispatch_kernel()(x, pos_a, pos_b, wa16, wb16)


def _sc_dispatch_body(x_hbm, pa_hbm, pb_hbm, wa_hbm, wb_hbm, xs_hbm, wts_hbm,
                      ia_v, ib_v, rows_v, wa_v, wb_v, sem):
    base = _sc_wid() * TPW

    def chunk(c, carry):
        t0 = base + c * DCHUNK
        pltpu.sync_copy(x_hbm.at[pl.ds(t0, DCHUNK)], rows_v)
        pltpu.sync_copy(pa_hbm.at[pl.ds(t0, DCHUNK)], ia_v)
        pltpu.sync_copy(pb_hbm.at[pl.ds(t0, DCHUNK)], ib_v)
        pltpu.sync_copy(wa_hbm.at[pl.ds(t0, DCHUNK)], wa_v)
        pltpu.sync_copy(wb_hbm.at[pl.ds(t0, DCHUNK)], wb_v)
        pltpu.async_copy(rows_v, xs_hbm.at[ia_v], sem).wait()
        pltpu.async_copy(rows_v, xs_hbm.at[ib_v], sem).wait()
        pltpu.async_copy(wa_v, wts_hbm.at[ia_v], sem).wait()
        pltpu.async_copy(wb_v, wts_hbm.at[ib_v], sem).wait()
        return carry

    lax.fori_loop(0, TPW // DCHUNK, chunk, 0)


@functools.cache
def _sc_combine_kernel():
    return functools.partial(
        pl.kernel,
        mesh=plsc.VectorSubcoreMesh(**_SC_MESH),
        out_type=jax.ShapeDtypeStruct((N, D), jnp.float32),
        scratch_types=[
            pltpu.VMEM((2 * CCHUNK,), jnp.int32),
            pltpu.VMEM((2 * CCHUNK, D), jnp.float32),
            pltpu.VMEM((CCHUNK, D), jnp.float32),
            pltpu.SemaphoreType.DMA,
        ],
    )(_sc_combine_body)


def _sc_combine(rows, pos_a, pos_b):
    return _sc_combine_kernel()(rows, pos_a, pos_b)


def _sc_combine_body(rows_hbm, pa_hbm, pb_hbm, out_hbm, idx_v, rows_v, acc_v,
                     sem):
    base = _sc_wid() * TPW

    def chunk(c, carry):
        t0 = base + c * CCHUNK
        pltpu.sync_copy(pa_hbm.at[pl.ds(t0, CCHUNK)],
                        idx_v.at[pl.ds(0, CCHUNK)])
        pltpu.sync_copy(pb_hbm.at[pl.ds(t0, CCHUNK)],
                        idx_v.at[pl.ds(CCHUNK, CCHUNK)])
        pltpu.async_copy(rows_hbm.at[idx_v], rows_v, sem).wait()

        def tok(i, c2):
            for j in range(D // 16):
                sl = pl.ds(j * 16, 16)
                acc_v[i, sl] = rows_v[i, sl] + rows_v[CCHUNK + i, sl]
            return c2

        lax.fori_loop(0, CCHUNK, tok, 0)
        pltpu.sync_copy(acc_v, out_hbm.at[pl.ds(t0, CCHUNK)])
        return carry

    lax.fori_loop(0, TPW // CCHUNK, chunk, 0)


def kernel(hidden_states, gate_W, w1, w3, w2):
    b, s, d = hidden_states.shape
    x = hidden_states.reshape(-1, d)
    utri = jnp.triu(jnp.ones((RBLK, RBLK), jnp.float32), 1)
    logits, ea, eb, wa16, wb16, ra, rb, counts = _router_call(x, gate_W, utri)
    counts = counts.astype(jnp.int32)
    padded = ((counts + TILE - 1) // TILE) * TILE
    pstart = jnp.concatenate([jnp.zeros((1,), jnp.int32),
                              jnp.cumsum(padded).astype(jnp.int32)])
    pos_a = pstart[ea] + ra
    pos_b = pstart[eb] + rb

    xs, wts16 = _sc_dispatch(x, pos_a, pos_b, wa16, wb16)
    outrows = _mlp_call(pstart, xs, w1, w3, w2, wts16)
    final = _sc_combine(outrows, pos_a, pos_b)
    return final.reshape(b, s, d), logits


# speedup vs baseline: 2.7597x; 1.0463x over previous
import functools

import jax
import jax.numpy as jnp
from jax import lax
from jax.experimental import pallas as pl
from jax.experimental.pallas import tpu as pltpu
from jax.experimental.pallas import tpu_sc as plsc

E = 8
TOPK = 2
D = 1024
I = 2048
N = 4096
A = N * TOPK
TILE = 256
PAD_A = A + E * TILE
NT = PAD_A // TILE
RBLK = 1024

_INTERPRET = False


def _router_body(x_ref, gw_ref, ut_ref, logits_ref, ea_ref, eb_ref, wa16_ref,
                 wb16_ref, ra_ref, rb_ref, counts_ref, carry_ref):
    t = pl.program_id(0)
    x = x_ref[...]
    logits_ref[...] = lax.dot_general(x, gw_ref[...], (((1,), (1,)), ((), ())),
                                      preferred_element_type=jnp.float32)
    lt = lax.dot_general(gw_ref[...], x, (((1,), (1,)), ((), ())),
                         preferred_element_type=jnp.float32)
    pt = jax.nn.softmax(lt, axis=0)
    iota_s = lax.broadcasted_iota(jnp.int32, (E, RBLK), 0)
    rank = jnp.zeros((E, RBLK), jnp.int32)
    for j in range(E):
        pj = pt[j:j + 1, :]
        beats = (pj > pt) | ((pj == pt) & (j < iota_s))
        rank = rank + beats.astype(jnp.int32)
    sel = rank < TOPK
    acc = jnp.zeros((1, RBLK), jnp.int32)
    rows = []
    for j in range(E):
        acc = acc + sel[j:j + 1, :].astype(jnp.int32)
        rows.append(acc)
    cnt = jnp.concatenate(rows, axis=0)
    first = sel & (cnt == 1)
    second = sel & (cnt == 2)
    ea = jnp.sum(jnp.where(first, iota_s, 0), axis=0, keepdims=True)
    eb = jnp.sum(jnp.where(second, iota_s, 0), axis=0, keepdims=True)
    ea_ref[...] = ea[0]
    eb_ref[...] = eb[0]
    wa = jnp.sum(jnp.where(first, pt, 0.0), axis=0)
    wb = jnp.sum(jnp.where(second, pt, 0.0), axis=0)
    ones_l = jnp.ones((1, 128), jnp.float32)
    wa16_ref[...] = wa[:, None] * ones_l
    wb16_ref[...] = wb[:, None] * ones_l
    self = sel.astype(jnp.float32)
    cumexcl = lax.dot_general(self, ut_ref[...], (((1,), (0,)), ((), ())),
                              preferred_element_type=jnp.float32)

    @pl.when(t == 0)
    def _():
        carry_ref[...] = jnp.zeros((E, 128), jnp.float32)

    carry = carry_ref[...][:, :1]
    cumexcl = cumexcl + carry
    ra_ref[...] = jnp.sum(
        jnp.where(iota_s == ea, cumexcl, 0.0), axis=0).astype(jnp.int32)
    rb_ref[...] = jnp.sum(
        jnp.where(iota_s == eb, cumexcl, 0.0), axis=0).astype(jnp.int32)
    new_carry = carry + jnp.sum(self, axis=1, keepdims=True)
    carry_ref[...] = new_carry * jnp.ones((1, 128), jnp.float32)
    counts_ref[...] = new_carry[:, 0]


def _router_call(x, gate_W, utri):
    return pl.pallas_call(
        _router_body,
        grid=(N // RBLK,),
        in_specs=[
            pl.BlockSpec((RBLK, D), lambda t: (t, 0)),
            pl.BlockSpec((E, D), lambda t: (0, 0)),
            pl.BlockSpec((RBLK, RBLK), lambda t: (0, 0)),
        ],
        out_specs=[
            pl.BlockSpec((RBLK, E), lambda t: (t, 0)),
            pl.BlockSpec((RBLK,), lambda t: (t,)),
            pl.BlockSpec((RBLK,), lambda t: (t,)),
            pl.BlockSpec((RBLK, 128), lambda t: (t, 0)),
            pl.BlockSpec((RBLK, 128), lambda t: (t, 0)),
            pl.BlockSpec((RBLK,), lambda t: (t,)),
            pl.BlockSpec((RBLK,), lambda t: (t,)),
            pl.BlockSpec((E,), lambda t: (0,)),
        ],
        out_shape=[
            jax.ShapeDtypeStruct((N, E), jnp.float32),
            jax.ShapeDtypeStruct((N,), jnp.int32),
            jax.ShapeDtypeStruct((N,), jnp.int32),
            jax.ShapeDtypeStruct((N, 128), jnp.float32),
            jax.ShapeDtypeStruct((N, 128), jnp.float32),
            jax.ShapeDtypeStruct((N,), jnp.int32),
            jax.ShapeDtypeStruct((N,), jnp.int32),
            jax.ShapeDtypeStruct((E,), jnp.float32),
        ],
        scratch_shapes=[pltpu.VMEM((E, 128), jnp.float32)],
        interpret=_INTERPRET,
    )(x, gate_W, utri)


def _tile_expert(t, ps):
    s = jnp.int32(0)
    for e in range(1, E + 1):
        s = s + (ps[e] <= t * TILE).astype(jnp.int32)
    return jnp.minimum(s, E - 1)


def _mlp_body(ps_ref, xs_ref, w1_ref, w3_ref, w2_ref, wt_ref, out_ref):
    t = pl.program_id(0)

    @pl.when(t * TILE < ps_ref[E])
    def _():
        xs = xs_ref[...]
        g = lax.dot_general(xs, w1_ref[...], (((1,), (1,)), ((), ())),
                            preferred_element_type=jnp.float32)
        u = lax.dot_general(xs, w3_ref[...], (((1,), (1,)), ((), ())),
                            preferred_element_type=jnp.float32)
        h = g * lax.logistic(g) * u
        o = lax.dot_general(h, w2_ref[...], (((1,), (1,)), ((), ())),
                            preferred_element_type=jnp.float32)
        out_ref[...] = o * wt_ref[:, :1]


def _mlp_call(pstart, xs, w1, w3, w2, wts16):
    grid_spec = pltpu.PrefetchScalarGridSpec(
        num_scalar_prefetch=1,
        grid=(NT,),
        in_specs=[
            pl.BlockSpec((TILE, D), lambda t, ps: (t, 0)),
            pl.BlockSpec((I, D), lambda t, ps: (_tile_expert(t, ps), 0)),
            pl.BlockSpec((I, D), lambda t, ps: (_tile_expert(t, ps), 0)),
            pl.BlockSpec((D, I), lambda t, ps: (0, _tile_expert(t, ps))),
            pl.BlockSpec((TILE, 128), lambda t, ps: (t, 0)),
        ],
        out_specs=pl.BlockSpec((TILE, D), lambda t, ps: (t, 0)),
    )
    return pl.pallas_call(
        _mlp_body,
        grid_spec=grid_spec,
        out_shape=jax.ShapeDtypeStruct((PAD_A, D), jnp.float32),
        interpret=_INTERPRET,
    )(pstart, xs, w1, w3, w2, wts16)


NW = 32
TPW = N // NW
DCHUNK = 64
CCHUNK = 32

_SC_MESH = dict(core_axis_name="c", subcore_axis_name="s")


def _sc_wid():
    return lax.axis_index("s") * 2 + lax.axis_index("c")


@functools.cache
def _sc_dispatch_kernel():
    return functools.partial(
        pl.kernel,
        mesh=plsc.VectorSubcoreMesh(**_SC_MESH),
        out_type=[
            jax.ShapeDtypeStruct((PAD_A, D), jnp.float32),
            jax.ShapeDtypeStruct((PAD_A, 128), jnp.float32),
        ],
        scratch_types=[
            pltpu.VMEM((DCHUNK,), jnp.int32),
            pltpu.VMEM((DCHUNK,), jnp.int32),
            pltpu.VMEM((DCHUNK, D), jnp.float32),
            pltpu.VMEM((DCHUNK, 128), jnp.float32),
            pltpu.VMEM((DCHUNK, 128), jnp.float32),
            pltpu.SemaphoreType.DMA,
        ],
    )(_sc_dispatch_body)


def _sc_dispatch(x, pos_a, pos_b, wa16, wb16):
    return _sc_dispatch_kernel()(x, pos_a, pos_b, wa16, wb16)


def _sc_dispatch_body(x_hbm, pa_hbm, pb_hbm, wa_hbm, wb_hbm, xs_hbm, wts_hbm,
                      ia_v, ib_v, rows_v, wa_v, wb_v, sem):
    base = _sc_wid() * TPW

    def chunk(c, carry):
        t0 = base + c * DCHUNK
        pltpu.sync_copy(x_hbm.at[pl.ds(t0, DCHUNK)], rows_v)
        pltpu.sync_copy(pa_hbm.at[pl.ds(t0, DCHUNK)], ia_v)
        pltpu.sync_copy(pb_hbm.at[pl.ds(t0, DCHUNK)], ib_v)
        pltpu.sync_copy(wa_hbm.at[pl.ds(t0, DCHUNK)], wa_v)
        pltpu.sync_copy(wb_hbm.at[pl.ds(t0, DCHUNK)], wb_v)
        pltpu.async_copy(rows_v, xs_hbm.at[ia_v], sem).wait()
        pltpu.async_copy(rows_v, xs_hbm.at[ib_v], sem).wait()
        pltpu.async_copy(wa_v, wts_hbm.at[ia_v], sem).wait()
        pltpu.async_copy(wb_v, wts_hbm.at[ib_v], sem).wait()
        return carry

    lax.fori_loop(0, TPW // DCHUNK, chunk, 0)


CB = 16


@functools.cache
def _sc_combine_kernel():
    return functools.partial(
        pl.kernel,
        mesh=plsc.VectorSubcoreMesh(**_SC_MESH),
        out_type=jax.ShapeDtypeStruct((N, D), jnp.float32),
        scratch_types=[
            pltpu.VMEM((TPW,), jnp.int32),
            pltpu.VMEM((TPW,), jnp.int32),
            pltpu.VMEM((CB, D), jnp.float32),
            pltpu.VMEM((CB, D), jnp.float32),
            pltpu.VMEM((CB, D), jnp.float32),
            pltpu.VMEM((CB, D), jnp.float32),
            pltpu.VMEM((CB, D), jnp.float32),
            pltpu.VMEM((CB, D), jnp.float32),
            pltpu.SemaphoreType.DMA,
            pltpu.SemaphoreType.DMA,
            pltpu.SemaphoreType.DMA,
            pltpu.SemaphoreType.DMA,
        ],
    )(_sc_combine_body)


def _sc_combine(rows, pos_a, pos_b):
    return _sc_combine_kernel()(rows, pos_a, pos_b)


def _sc_combine_body(rows_hbm, pa_hbm, pb_hbm, out_hbm, ia_v, ib_v,
                     ra0, ra1, rb0, rb1, ac0, ac1, g0, g1, w0, w1):
    base = _sc_wid() * TPW
    pltpu.sync_copy(pa_hbm.at[pl.ds(base, TPW)], ia_v)
    pltpu.sync_copy(pb_hbm.at[pl.ds(base, TPW)], ib_v)
    ra, rb, ac = [ra0, ra1], [rb0, rb1], [ac0, ac1]
    gsem, wsem = [g0, g1], [w0, w1]
    nch = TPW // CB
    handles = [None, None]
    wh = [None, None]

    def start(c):
        s = c % 2
        handles[s] = (
            pltpu.async_copy(rows_hbm.at[ia_v.at[pl.ds(c * CB, CB)]],
                             ra[s], gsem[s]),
            pltpu.async_copy(rows_hbm.at[ib_v.at[pl.ds(c * CB, CB)]],
                             rb[s], gsem[s]),
        )

    start(0)
    for c in range(nch):
        s = c % 2
        if c + 1 < nch:
            start(c + 1)
        ha, hb = handles[s]
        ha.wait()
        hb.wait()
        if wh[s] is not None:
            wh[s].wait()
        ras, rbs, acs = ra[s], rb[s], ac[s]

        def tok(i, c2):
            for j in range(D // 16):
                sl = pl.ds(j * 16, 16)
                acs[i, sl] = ras[i, sl] + rbs[i, sl]
            return c2

        lax.fori_loop(0, CB, tok, 0)
        wh[s] = pltpu.async_copy(acs, out_hbm.at[pl.ds(base + c * CB, CB)],
                                 wsem[s])
    for h in wh:
        if h is not None:
            h.wait()


def kernel(hidden_states, gate_W, w1, w3, w2):
    b, s, d = hidden_states.shape
    x = hidden_states.reshape(-1, d)
    utri = jnp.triu(jnp.ones((RBLK, RBLK), jnp.float32), 1)
    logits, ea, eb, wa16, wb16, ra, rb, counts = _router_call(x, gate_W, utri)
    counts = counts.astype(jnp.int32)
    padded = ((counts + TILE - 1) // TILE) * TILE
    pstart = jnp.concatenate([jnp.zeros((1,), jnp.int32),
                              jnp.cumsum(padded).astype(jnp.int32)])
    pos_a = pstart[ea] + ra
    pos_b = pstart[eb] + rb

    xs, wts16 = _sc_dispatch(x, pos_a, pos_b, wa16, wb16)
    outrows = _mlp_call(pstart, xs, w1, w3, w2, wts16)
    final = _sc_combine(outrows, pos_a, pos_b)
    return final.reshape(b, s, d), logits


# speedup vs baseline: 2.7605x; 1.0003x over previous
import functools

import jax
import jax.numpy as jnp
from jax import lax
from jax.experimental import pallas as pl
from jax.experimental.pallas import tpu as pltpu
from jax.experimental.pallas import tpu_sc as plsc

E = 8
TOPK = 2
D = 1024
I = 2048
N = 4096
A = N * TOPK
TILE = 256
PAD_A = A + E * TILE
NT = PAD_A // TILE
RBLK = 1024


def _router_body(x_ref, gw_ref, ut_ref, logits_ref, ea_ref, eb_ref, wa16_ref,
                 wb16_ref, ra_ref, rb_ref, counts_ref, carry_ref):
    t = pl.program_id(0)
    x = x_ref[...]
    logits_ref[...] = lax.dot_general(x, gw_ref[...], (((1,), (1,)), ((), ())),
                                      preferred_element_type=jnp.float32)
    lt = lax.dot_general(gw_ref[...], x, (((1,), (1,)), ((), ())),
                         preferred_element_type=jnp.float32)
    pt = jax.nn.softmax(lt, axis=0)
    iota_s = lax.broadcasted_iota(jnp.int32, (E, RBLK), 0)
    rank = jnp.zeros((E, RBLK), jnp.int32)
    for j in range(E):
        pj = pt[j:j + 1, :]
        beats = (pj > pt) | ((pj == pt) & (j < iota_s))
        rank = rank + beats.astype(jnp.int32)
    sel = rank < TOPK
    acc = jnp.zeros((1, RBLK), jnp.int32)
    rows = []
    for j in range(E):
        acc = acc + sel[j:j + 1, :].astype(jnp.int32)
        rows.append(acc)
    cnt = jnp.concatenate(rows, axis=0)
    first = sel & (cnt == 1)
    second = sel & (cnt == 2)
    ea = jnp.sum(jnp.where(first, iota_s, 0), axis=0, keepdims=True)
    eb = jnp.sum(jnp.where(second, iota_s, 0), axis=0, keepdims=True)
    ea_ref[...] = ea[0]
    eb_ref[...] = eb[0]
    wa = jnp.sum(jnp.where(first, pt, 0.0), axis=0)
    wb = jnp.sum(jnp.where(second, pt, 0.0), axis=0)
    ones_l = jnp.ones((1, 128), jnp.float32)
    wa16_ref[...] = wa[:, None] * ones_l
    wb16_ref[...] = wb[:, None] * ones_l
    self = sel.astype(jnp.float32)
    cumexcl = lax.dot_general(self, ut_ref[...], (((1,), (0,)), ((), ())),
                              preferred_element_type=jnp.float32)

    @pl.when(t == 0)
    def _():
        carry_ref[...] = jnp.zeros((E, 128), jnp.float32)

    carry = carry_ref[...][:, :1]
    cumexcl = cumexcl + carry
    ra_ref[...] = jnp.sum(
        jnp.where(iota_s == ea, cumexcl, 0.0), axis=0).astype(jnp.int32)
    rb_ref[...] = jnp.sum(
        jnp.where(iota_s == eb, cumexcl, 0.0), axis=0).astype(jnp.int32)
    new_carry = carry + jnp.sum(self, axis=1, keepdims=True)
    carry_ref[...] = new_carry * jnp.ones((1, 128), jnp.float32)
    counts_ref[...] = new_carry[:, 0]


def _router_call(x, gate_W, utri):
    return pl.pallas_call(
        _router_body,
        grid=(N // RBLK,),
        in_specs=[
            pl.BlockSpec((RBLK, D), lambda t: (t, 0)),
            pl.BlockSpec((E, D), lambda t: (0, 0)),
            pl.BlockSpec((RBLK, RBLK), lambda t: (0, 0)),
        ],
        out_specs=[
            pl.BlockSpec((RBLK, E), lambda t: (t, 0)),
            pl.BlockSpec((RBLK,), lambda t: (t,)),
            pl.BlockSpec((RBLK,), lambda t: (t,)),
            pl.BlockSpec((RBLK, 128), lambda t: (t, 0)),
            pl.BlockSpec((RBLK, 128), lambda t: (t, 0)),
            pl.BlockSpec((RBLK,), lambda t: (t,)),
            pl.BlockSpec((RBLK,), lambda t: (t,)),
            pl.BlockSpec((E,), lambda t: (0,)),
        ],
        out_shape=[
            jax.ShapeDtypeStruct((N, E), jnp.float32),
            jax.ShapeDtypeStruct((N,), jnp.int32),
            jax.ShapeDtypeStruct((N,), jnp.int32),
            jax.ShapeDtypeStruct((N, 128), jnp.float32),
            jax.ShapeDtypeStruct((N, 128), jnp.float32),
            jax.ShapeDtypeStruct((N,), jnp.int32),
            jax.ShapeDtypeStruct((N,), jnp.int32),
            jax.ShapeDtypeStruct((E,), jnp.float32),
        ],
        scratch_shapes=[pltpu.VMEM((E, 128), jnp.float32)],
    )(x, gate_W, utri)


def _tile_expert(t, ps):
    s = jnp.int32(0)
    for e in range(1, E + 1):
        s = s + (ps[e] <= t * TILE).astype(jnp.int32)
    return jnp.minimum(s, E - 1)


def _mlp_body(ps_ref, xs_ref, w1_ref, w3_ref, w2_ref, wt_ref, out_ref):
    t = pl.program_id(0)

    @pl.when(t * TILE < ps_ref[E])
    def _():
        xs = xs_ref[...]
        g = lax.dot_general(xs, w1_ref[...], (((1,), (1,)), ((), ())),
                            preferred_element_type=jnp.float32)
        u = lax.dot_general(xs, w3_ref[...], (((1,), (1,)), ((), ())),
                            preferred_element_type=jnp.float32)
        h = g * lax.logistic(g) * u
        o = lax.dot_general(h, w2_ref[...], (((1,), (1,)), ((), ())),
                            preferred_element_type=jnp.float32)
        out_ref[...] = o * wt_ref[:, :1]


def _mlp_call(pstart, xs, w1, w3, w2, wts16):
    grid_spec = pltpu.PrefetchScalarGridSpec(
        num_scalar_prefetch=1,
        grid=(NT,),
        in_specs=[
            pl.BlockSpec((TILE, D), lambda t, ps: (t, 0)),
            pl.BlockSpec((I, D), lambda t, ps: (_tile_expert(t, ps), 0)),
            pl.BlockSpec((I, D), lambda t, ps: (_tile_expert(t, ps), 0)),
            pl.BlockSpec((D, I), lambda t, ps: (0, _tile_expert(t, ps))),
            pl.BlockSpec((TILE, 128), lambda t, ps: (t, 0)),
        ],
        out_specs=pl.BlockSpec((TILE, D), lambda t, ps: (t, 0)),
    )
    return pl.pallas_call(
        _mlp_body,
        grid_spec=grid_spec,
        out_shape=jax.ShapeDtypeStruct((PAD_A, D), jnp.float32),
    )(pstart, xs, w1, w3, w2, wts16)


NW = 32
TPW = N // NW
DCHUNK = 64
CCHUNK = 32

_SC_MESH = dict(core_axis_name="c", subcore_axis_name="s")


def _sc_wid():
    return lax.axis_index("s") * 2 + lax.axis_index("c")


@functools.cache
def _sc_dispatch_kernel():
    return functools.partial(
        pl.kernel,
        mesh=plsc.VectorSubcoreMesh(**_SC_MESH),
        out_type=[
            jax.ShapeDtypeStruct((PAD_A, D), jnp.float32),
            jax.ShapeDtypeStruct((PAD_A, 128), jnp.float32),
        ],
        scratch_types=[
            pltpu.VMEM((DCHUNK,), jnp.int32),
            pltpu.VMEM((DCHUNK,), jnp.int32),
            pltpu.VMEM((DCHUNK, D), jnp.float32),
            pltpu.VMEM((DCHUNK, 128), jnp.float32),
            pltpu.VMEM((DCHUNK, 128), jnp.float32),
            pltpu.SemaphoreType.DMA,
        ],
    )(_sc_dispatch_body)


def _sc_dispatch(x, pos_a, pos_b, wa16, wb16):
    return _sc_dispatch_kernel()(x, pos_a, pos_b, wa16, wb16)


def _sc_dispatch_body(x_hbm, pa_hbm, pb_hbm, wa_hbm, wb_hbm, xs_hbm, wts_hbm,
                      ia_v, ib_v, rows_v, wa_v, wb_v, sem):
    base = _sc_wid() * TPW

    def chunk(c, carry):
        t0 = base + c * DCHUNK
        pltpu.sync_copy(x_hbm.at[pl.ds(t0, DCHUNK)], rows_v)
        pltpu.sync_copy(pa_hbm.at[pl.ds(t0, DCHUNK)], ia_v)
        pltpu.sync_copy(pb_hbm.at[pl.ds(t0, DCHUNK)], ib_v)
        pltpu.sync_copy(wa_hbm.at[pl.ds(t0, DCHUNK)], wa_v)
        pltpu.sync_copy(wb_hbm.at[pl.ds(t0, DCHUNK)], wb_v)
        pltpu.async_copy(rows_v, xs_hbm.at[ia_v], sem).wait()
        pltpu.async_copy(rows_v, xs_hbm.at[ib_v], sem).wait()
        pltpu.async_copy(wa_v, wts_hbm.at[ia_v], sem).wait()
        pltpu.async_copy(wb_v, wts_hbm.at[ib_v], sem).wait()
        return carry

    lax.fori_loop(0, TPW // DCHUNK, chunk, 0)


CB = 16


@functools.cache
def _sc_combine_kernel():
    return functools.partial(
        pl.kernel,
        mesh=plsc.VectorSubcoreMesh(**_SC_MESH),
        out_type=jax.ShapeDtypeStruct((N, D), jnp.float32),
        scratch_types=[
            pltpu.VMEM((TPW,), jnp.int32),
            pltpu.VMEM((TPW,), jnp.int32),
            pltpu.VMEM((CB, D), jnp.float32),
            pltpu.VMEM((CB, D), jnp.float32),
            pltpu.VMEM((CB, D), jnp.float32),
            pltpu.VMEM((CB, D), jnp.float32),
            pltpu.VMEM((CB, D), jnp.float32),
            pltpu.VMEM((CB, D), jnp.float32),
            pltpu.SemaphoreType.DMA,
            pltpu.SemaphoreType.DMA,
            pltpu.SemaphoreType.DMA,
            pltpu.SemaphoreType.DMA,
        ],
    )(_sc_combine_body)


def _sc_combine(rows, pos_a, pos_b):
    return _sc_combine_kernel()(rows, pos_a, pos_b)


def _sc_combine_body(rows_hbm, pa_hbm, pb_hbm, out_hbm, ia_v, ib_v,
                     ra0, ra1, rb0, rb1, ac0, ac1, g0, g1, w0, w1):
    base = _sc_wid() * TPW
    pltpu.sync_copy(pa_hbm.at[pl.ds(base, TPW)], ia_v)
    pltpu.sync_copy(pb_hbm.at[pl.ds(base, TPW)], ib_v)
    ra, rb, ac = [ra0, ra1], [rb0, rb1], [ac0, ac1]
    gsem, wsem = [g0, g1], [w0, w1]
    nch = TPW // CB
    handles = [None, None]
    wh = [None, None]

    def start(c):
        s = c % 2
        handles[s] = (
            pltpu.async_copy(rows_hbm.at[ia_v.at[pl.ds(c * CB, CB)]],
                             ra[s], gsem[s]),
            pltpu.async_copy(rows_hbm.at[ib_v.at[pl.ds(c * CB, CB)]],
                             rb[s], gsem[s]),
        )

    start(0)
    for c in range(nch):
        s = c % 2
        if c + 1 < nch:
            start(c + 1)
        ha, hb = handles[s]
        ha.wait()
        hb.wait()
        if wh[s] is not None:
            wh[s].wait()
        ras, rbs, acs = ra[s], rb[s], ac[s]

        def tok(i, c2):
            for j in range(D // 16):
                sl = pl.ds(j * 16, 16)
                acs[i, sl] = ras[i, sl] + rbs[i, sl]
            return c2

        lax.fori_loop(0, CB, tok, 0)
        wh[s] = pltpu.async_copy(acs, out_hbm.at[pl.ds(base + c * CB, CB)],
                                 wsem[s])
    for h in wh:
        if h is not None:
            h.wait()


def kernel(hidden_states, gate_W, w1, w3, w2):
    b, s, d = hidden_states.shape
    x = hidden_states.reshape(-1, d)
    utri = jnp.triu(jnp.ones((RBLK, RBLK), jnp.float32), 1)
    logits, ea, eb, wa16, wb16, ra, rb, counts = _router_call(x, gate_W, utri)
    counts = counts.astype(jnp.int32)
    padded = ((counts + TILE - 1) // TILE) * TILE
    pstart = jnp.concatenate([jnp.zeros((1,), jnp.int32),
                              jnp.cumsum(padded).astype(jnp.int32)])
    pos_a = pstart[ea] + ra
    pos_b = pstart[eb] + rb

    xs, wts16 = _sc_dispatch(x, pos_a, pos_b, wa16, wb16)
    outrows = _mlp_call(pstart, xs, w1, w3, w2, wts16)
    final = _sc_combine(outrows, pos_a, pos_b)
    return final.reshape(b, s, d), logits
